# baseline (speedup 1.0000x reference)
import functools

import jax
import jax.numpy as jnp
from jax import lax
from jax.experimental import pallas as pl
from jax.experimental.pallas import tpu as pltpu
from jax.experimental.pallas import tpu_sc as plsc

N1, N2, Q, E, DC = 8, 4, 8192, 65536, 128
HALF = DC // 2
R1 = N1 * Q
R2 = N2 * E
ROWS = R1 + R2
NC, NS = 2, 16
NW = NC * NS
QB = 512

H_TOTAL = 2 * R2
H_PER_W = H_TOTAL // NW
CH = 512
CR = CH // 2
NCHUNK = H_PER_W // CH
WPG = NW // N2


def _table_body(gset2_ref, qpair_ref, g2_ref, out_ref):
    qp = qpair_ref[...]
    for gj in range(2 * N2):
        gi = gset2_ref[gj // 2]
        row = g2_ref[pl.ds(gi, 1), 0]
        j = gj % 2
        half = row[:, j * HALF:(j + 1) * HALF]
        bias = jnp.concatenate([half, half], axis=-1)
        out_ref[gj] = qp + bias


def _build_table(gset_2q, qpair, G2):
    qb = Q // 16
    return pl.pallas_call(
        _table_body,
        grid_spec=pltpu.PrefetchScalarGridSpec(
            num_scalar_prefetch=1,
            grid=(Q // 2 // qb,),
            in_specs=[
                pl.BlockSpec((qb, DC), lambda q, gset: (q, 0)),
                pl.BlockSpec((16, 1, DC), lambda q, gset: (0, 0, 0)),
            ],
            out_specs=pl.BlockSpec(
                (2 * N2, qb, DC), lambda q, gset: (0, q, 0)
            ),
        ),
        out_shape=jax.ShapeDtypeStruct((2 * N2, Q // 2, DC), jnp.float32),
    )(gset_2q, qpair, G2)


E_PER_W = R2 // NW
CE = 256
NCE = E_PER_W // CE


def _sc_body(table_hbm, lay3_hbm, out_hbm, idx_v, data_v, sem_g, sem_out):
    w = lax.axis_index("s") * NC + lax.axis_index("c")
    g = w // WPG
    ebase = (w % WPG) * E_PER_W
    tbl = [
        table_hbm.at[pl.ds(pl.multiple_of((2 * g + tc) * Q, Q), Q)]
        for tc in range(2)
    ]

    def chunk(c, p):
        e0 = ebase + c * CE
        row0 = R1 + g * E + e0
        dsts = [out_hbm.at[pl.ds(row0, CE), tc] for tc in range(2)]

        @pl.when(c >= 2)
        def _():
            for tc in range(2):
                pltpu.make_async_copy(
                    data_v.at[p, tc], dsts[tc], sem_out.at[p]
                ).wait()

        m0 = pl.multiple_of(e0 // 128, 2)
        pltpu.sync_copy(lay3_hbm.at[pl.ds(m0, 2)], idx_v)
        copies = [
            pltpu.async_copy(
                tbl[tc].at[idx_v.at[m2, tc]],
                data_v.at[p, tc, pl.ds(m2 * 128, 128)],
                sem_g,
            )
            for m2 in range(2)
            for tc in range(2)
        ]
        for cp in copies:
            cp.wait()
        for tc in range(2):
            pltpu.async_copy(data_v.at[p, tc], dsts[tc], sem_out.at[p])

    def pair(i, _):
        chunk(2 * i, 0)
        chunk(2 * i + 1, 1)
        return ()

    lax.fori_loop(0, NCE // 2, pair, (), unroll=False)

    for p, c in ((0, NCE - 2), (1, NCE - 1)):
        e0 = ebase + c * CE
        row0 = R1 + g * E + e0
        for tc in range(2):
            pltpu.make_async_copy(
                data_v.at[p, tc],
                out_hbm.at[pl.ds(row0, CE), tc],
                sem_out.at[p],
            ).wait()


def _sc_gather(table, lay3):
    mesh = plsc.VectorSubcoreMesh(
        core_axis_name="c", subcore_axis_name="s", num_cores=NC, num_subcores=NS
    )
    f = functools.partial(
        pl.kernel,
        out_type=jax.ShapeDtypeStruct((ROWS, 2, HALF), jnp.float32),
        mesh=mesh,
        scratch_types=[
            pltpu.VMEM((2, 2, 128), jnp.int32),
            pltpu.VMEM((2, 2, CE, HALF), jnp.float32),
            pltpu.SemaphoreType.DMA,
            pltpu.SemaphoreType.DMA((2,)),
        ],
        compiler_params=pltpu.CompilerParams(use_tc_tiling_on_sc=False),
    )(_sc_body)
    return f(table, lay3)


def _tok1_body(gset1_ref, prev_ref, qub_ref, g1_ref, out_ref):
    del gset1_ref, prev_ref
    out_ref[...] = qub_ref[...] + g1_ref[0]


def _fill_tok1(gset_1q, prev, qubits, G1):
    return pl.pallas_call(
        _tok1_body,
        grid_spec=pltpu.PrefetchScalarGridSpec(
            num_scalar_prefetch=1,
            grid=(N1,),
            in_specs=[
                pl.BlockSpec(memory_space=pl.ANY),
                pl.BlockSpec((Q, DC), lambda g, gset: (0, 0)),
                pl.BlockSpec((1, 1, DC), lambda g, gset: (gset[g], 0, 0)),
            ],
            out_specs=pl.BlockSpec((Q, DC), lambda g, gset: (g, 0)),
        ),
        out_shape=jax.ShapeDtypeStruct((ROWS, DC), jnp.float32),
        input_output_aliases={1: 0},
    )(gset_1q, prev, qubits, G1[:, None, :])


def kernel(gset_1q, gset_2q, qubits, layout, G1, G2):
    qpair = qubits[:, :HALF].reshape(Q // 2, DC)
    table = _build_table(gset_2q, qpair, G2[:, None, :]).reshape(2 * N2 * Q, HALF)
    lay3 = layout.reshape(E // 128, 128, 2).transpose(0, 2, 1)
    out = _sc_gather(table, lay3).reshape(ROWS, DC)
    return _fill_tok1(gset_1q, out, qubits, G1)


# speedup vs baseline: 8.4445x; 8.4445x over previous
import functools

import jax
import jax.numpy as jnp
from jax import lax
from jax.experimental import pallas as pl
from jax.experimental.pallas import tpu as pltpu
from jax.experimental.pallas import tpu_sc as plsc

N1, N2, Q, E, DC = 8, 4, 8192, 65536, 128
HALF = DC // 2
R1 = N1 * Q
R2 = N2 * E
ROWS = R1 + R2
NC, NS = 2, 16
NW = NC * NS
QB = 512

H_TOTAL = 2 * R2
H_PER_W = H_TOTAL // NW
CH = 512
CR = CH // 2
NCHUNK = H_PER_W // CH
WPG = NW // N2


def _table_body(gset2_ref, qpair_ref, g2_ref, out_ref):
    qp = qpair_ref[...]
    for gj in range(2 * N2):
        gi = gset2_ref[gj // 2]
        row = g2_ref[pl.ds(gi, 1), 0]
        j = gj % 2
        half = row[:, j * HALF:(j + 1) * HALF]
        bias = jnp.concatenate([half, half], axis=-1)
        out_ref[gj] = qp + bias


def _build_table(gset_2q, qpair, G2):
    qb = Q // 16
    return pl.pallas_call(
        _table_body,
        grid_spec=pltpu.PrefetchScalarGridSpec(
            num_scalar_prefetch=1,
            grid=(Q // 2 // qb,),
            in_specs=[
                pl.BlockSpec((qb, DC), lambda q, gset: (q, 0)),
                pl.BlockSpec((16, 1, DC), lambda q, gset: (0, 0, 0)),
            ],
            out_specs=pl.BlockSpec(
                (2 * N2, qb, DC), lambda q, gset: (0, q, 0)
            ),
        ),
        out_shape=jax.ShapeDtypeStruct((2 * N2, Q // 2, DC), jnp.float32),
    )(gset_2q, qpair, G2)


E_PER_W = R2 // NW
CE = 256
NCE = E_PER_W // CE


def _sc_body(table_hbm, lay3_hbm, out_hbm, idx_raw, idx_v, data_v, sem_g, sem_out):
    w = lax.axis_index("s") * NC + lax.axis_index("c")
    g = w // WPG
    ebase = (w % WPG) * E_PER_W
    lane = lax.iota(jnp.int32, 16)
    off = [(2 * g + tc) * Q for tc in range(2)]

    def chunk(c, p):
        e0 = ebase + c * CE
        hb0 = (R1 + g * E + e0) // 64
        dst = out_hbm.at[pl.ds(hb0, 4)]

        @pl.when(c >= 2)
        def _():
            pltpu.make_async_copy(data_v.at[p], dst, sem_out.at[p]).wait()

        copies = [
            pltpu.async_copy(
                table_hbm.at[idx_v.at[4 * p + k]], data_v.at[p, k], sem_g
            )
            for k in range(4)
        ]
        for cp in copies:
            cp.wait()
        pltpu.async_copy(data_v.at[p], dst, sem_out.at[p])

    def pair(i, _):
        m0 = pl.multiple_of((ebase + 2 * i * CE) // 128, 4)
        pltpu.sync_copy(lay3_hbm.at[pl.ds(m0, 4)], idx_raw)
        for m in range(4):
            for tc in range(2):
                for s in range(8):
                    vals = idx_raw[m, tc, pl.ds(16 * s, 16)] + off[tc]
                    row = 2 * m + (1 if s >= 4 else 0)
                    col = 2 * lane + ((32 * s + tc) % 128)
                    plsc.store_scatter(
                        idx_v, [jnp.full((16,), row, jnp.int32), col], vals
                    )
        chunk(2 * i, 0)
        chunk(2 * i + 1, 1)
        return ()

    lax.fori_loop(0, NCE // 2, pair, (), unroll=False)

    for p, c in ((0, NCE - 2), (1, NCE - 1)):
        e0 = ebase + c * CE
        hb0 = (R1 + g * E + e0) // 64
        pltpu.make_async_copy(
            data_v.at[p], out_hbm.at[pl.ds(hb0, 4)], sem_out.at[p]
        ).wait()


def _sc_gather(table, lay3):
    mesh = plsc.VectorSubcoreMesh(
        core_axis_name="c", subcore_axis_name="s", num_cores=NC, num_subcores=NS
    )
    f = functools.partial(
        pl.kernel,
        out_type=jax.ShapeDtypeStruct((ROWS * DC // (128 * HALF), 128, HALF), jnp.float32),
        mesh=mesh,
        scratch_types=[
            pltpu.VMEM((4, 2, 128), jnp.int32),
            pltpu.VMEM((8, 128), jnp.int32),
            pltpu.VMEM((2, 4, 128, HALF), jnp.float32),
            pltpu.SemaphoreType.DMA,
            pltpu.SemaphoreType.DMA((2,)),
        ],
        compiler_params=pltpu.CompilerParams(
            use_tc_tiling_on_sc=False, needs_layout_passes=False
        ),
    )(_sc_body)
    return f(table, lay3)


def _tok1_body(gset1_ref, prev_ref, qub_ref, g1_ref, out_ref):
    del gset1_ref, prev_ref
    out_ref[...] = qub_ref[...] + g1_ref[0]


def _fill_tok1(gset_1q, prev, qubits, G1):
    return pl.pallas_call(
        _tok1_body,
        grid_spec=pltpu.PrefetchScalarGridSpec(
            num_scalar_prefetch=1,
            grid=(N1,),
            in_specs=[
                pl.BlockSpec(memory_space=pl.ANY),
                pl.BlockSpec((Q, DC), lambda g, gset: (0, 0)),
                pl.BlockSpec((1, 1, DC), lambda g, gset: (gset[g], 0, 0)),
            ],
            out_specs=pl.BlockSpec((Q, DC), lambda g, gset: (g, 0)),
        ),
        out_shape=jax.ShapeDtypeStruct((ROWS, DC), jnp.float32),
        input_output_aliases={1: 0},
    )(gset_1q, prev, qubits, G1[:, None, :])


def kernel(gset_1q, gset_2q, qubits, layout, G1, G2):
    qpair = qubits[:, :HALF].reshape(Q // 2, DC)
    table = _build_table(gset_2q, qpair, G2[:, None, :]).reshape(2 * N2 * Q, HALF)
    lay3 = layout.reshape(E // 128, 128, 2).transpose(0, 2, 1)
    out = _sc_gather(table, lay3).reshape(ROWS, DC)
    return _fill_tok1(gset_1q, out, qubits, G1)


# speedup vs baseline: 8.7382x; 1.0348x over previous
import functools

import jax
import jax.numpy as jnp
from jax import lax
from jax.experimental import pallas as pl
from jax.experimental.pallas import tpu as pltpu
from jax.experimental.pallas import tpu_sc as plsc

N1, N2, Q, E, DC = 8, 4, 8192, 65536, 128
HALF = DC // 2
R1 = N1 * Q
R2 = N2 * E
ROWS = R1 + R2
NC, NS = 2, 16
NW = NC * NS
QB = 512

H_TOTAL = 2 * R2
H_PER_W = H_TOTAL // NW
CH = 512
CR = CH // 2
NCHUNK = H_PER_W // CH
WPG = NW // N2


def _table_body(gset2_ref, qpair_ref, g2_ref, out_ref):
    qp = qpair_ref[...]
    for gj in range(2 * N2):
        gi = gset2_ref[gj // 2]
        row = g2_ref[pl.ds(gi, 1), 0]
        j = gj % 2
        half = row[:, j * HALF:(j + 1) * HALF]
        bias = jnp.concatenate([half, half], axis=-1)
        out_ref[gj] = qp + bias


def _build_table(gset_2q, qpair, G2):
    qb = Q // 16
    return pl.pallas_call(
        _table_body,
        grid_spec=pltpu.PrefetchScalarGridSpec(
            num_scalar_prefetch=1,
            grid=(Q // 2 // qb,),
            in_specs=[
                pl.BlockSpec((qb, DC), lambda q, gset: (q, 0)),
                pl.BlockSpec((16, 1, DC), lambda q, gset: (0, 0, 0)),
            ],
            out_specs=pl.BlockSpec(
                (2 * N2, qb, DC), lambda q, gset: (0, q, 0)
            ),
        ),
        out_shape=jax.ShapeDtypeStruct((2 * N2, Q // 2, DC), jnp.float32),
    )(gset_2q, qpair, G2)


E_PER_W = R2 // NW
CE = 256
NCE = E_PER_W // CE


def _sc_body(table_hbm, lay3_hbm, out_hbm, idx_raw, idx_v, data_v, sem_g, sem_out):
    w = lax.axis_index("s") * NC + lax.axis_index("c")
    g = w // WPG
    ebase = (w % WPG) * E_PER_W
    lane = lax.iota(jnp.int32, 16)
    off = [(2 * g + tc) * Q for tc in range(2)]

    def chunk(c, p):
        e0 = ebase + c * CE
        hb0 = (R1 + g * E + e0) // 64
        dst = out_hbm.at[pl.ds(hb0, 4)]

        @pl.when(c >= 2)
        def _():
            pltpu.make_async_copy(data_v.at[p], dst, sem_out.at[p]).wait()

        copies = [
            pltpu.async_copy(
                table_hbm.at[idx_v.at[4 * p + k]], data_v.at[p, k], sem_g
            )
            for k in range(4)
        ]
        for cp in copies:
            cp.wait()
        pltpu.async_copy(data_v.at[p], dst, sem_out.at[p])

    m0w = pl.multiple_of(ebase // 128, 64)
    pltpu.sync_copy(lay3_hbm.at[pl.ds(m0w, E_PER_W // 128)], idx_raw)

    def pair(i, _):
        for m in range(4):
            for tc in range(2):
                for s in range(8):
                    vals = idx_raw[4 * i + m, tc, pl.ds(16 * s, 16)] + off[tc]
                    row = 2 * m + (1 if s >= 4 else 0)
                    col = 2 * lane + ((32 * s + tc) % 128)
                    plsc.store_scatter(
                        idx_v, [jnp.full((16,), row, jnp.int32), col], vals
                    )
        chunk(2 * i, 0)
        chunk(2 * i + 1, 1)
        return ()

    lax.fori_loop(0, NCE // 2, pair, (), unroll=False)

    for p, c in ((0, NCE - 2), (1, NCE - 1)):
        e0 = ebase + c * CE
        hb0 = (R1 + g * E + e0) // 64
        pltpu.make_async_copy(
            data_v.at[p], out_hbm.at[pl.ds(hb0, 4)], sem_out.at[p]
        ).wait()


def _sc_gather(table, lay3):
    mesh = plsc.VectorSubcoreMesh(
        core_axis_name="c", subcore_axis_name="s", num_cores=NC, num_subcores=NS
    )
    f = functools.partial(
        pl.kernel,
        out_type=jax.ShapeDtypeStruct((ROWS * DC // (128 * HALF), 128, HALF), jnp.float32),
        mesh=mesh,
        scratch_types=[
            pltpu.VMEM((E_PER_W // 128, 2, 128), jnp.int32),
            pltpu.VMEM((8, 128), jnp.int32),
            pltpu.VMEM((2, 4, 128, HALF), jnp.float32),
            pltpu.SemaphoreType.DMA,
            pltpu.SemaphoreType.DMA((2,)),
        ],
        compiler_params=pltpu.CompilerParams(
            use_tc_tiling_on_sc=False, needs_layout_passes=False
        ),
    )(_sc_body)
    return f(table, lay3)


def _tok1_body(gset1_ref, prev_ref, qub_ref, g1_ref, out_ref):
    del gset1_ref, prev_ref
    out_ref[...] = qub_ref[...] + g1_ref[0]


def _fill_tok1(gset_1q, prev, qubits, G1):
    return pl.pallas_call(
        _tok1_body,
        grid_spec=pltpu.PrefetchScalarGridSpec(
            num_scalar_prefetch=1,
            grid=(N1,),
            in_specs=[
                pl.BlockSpec(memory_space=pl.ANY),
                pl.BlockSpec((Q, DC), lambda g, gset: (0, 0)),
                pl.BlockSpec((1, 1, DC), lambda g, gset: (gset[g], 0, 0)),
            ],
            out_specs=pl.BlockSpec((Q, DC), lambda g, gset: (g, 0)),
        ),
        out_shape=jax.ShapeDtypeStruct((ROWS, DC), jnp.float32),
        input_output_aliases={1: 0},
    )(gset_1q, prev, qubits, G1[:, None, :])


def kernel(gset_1q, gset_2q, qubits, layout, G1, G2):
    qpair = qubits[:, :HALF].reshape(Q // 2, DC)
    table = _build_table(gset_2q, qpair, G2[:, None, :]).reshape(2 * N2 * Q, HALF)
    lay3 = layout.reshape(E // 128, 128, 2).transpose(0, 2, 1)
    out = _sc_gather(table, lay3).reshape(ROWS, DC)
    return _fill_tok1(gset_1q, out, qubits, G1)


# speedup vs baseline: 8.7413x; 1.0004x over previous
import functools

import jax
import jax.numpy as jnp
from jax import lax
from jax.experimental import pallas as pl
from jax.experimental.pallas import tpu as pltpu
from jax.experimental.pallas import tpu_sc as plsc

N1, N2, Q, E, DC = 8, 4, 8192, 65536, 128
HALF = DC // 2
R1 = N1 * Q
R2 = N2 * E
ROWS = R1 + R2
NC, NS = 2, 16
NW = NC * NS
QB = 512

H_TOTAL = 2 * R2
H_PER_W = H_TOTAL // NW
CH = 512
CR = CH // 2
NCHUNK = H_PER_W // CH
WPG = NW // N2


def _table_body(gset2_ref, qpair_ref, g2_ref, out_ref):
    blk = qpair_ref[...]
    qp = jnp.concatenate(
        [blk[:, :HALF], blk[:, 2 * HALF:3 * HALF]], axis=-1
    )
    for gj in range(2 * N2):
        gi = gset2_ref[gj // 2]
        row = g2_ref[pl.ds(gi, 1), 0]
        j = gj % 2
        half = row[:, j * HALF:(j + 1) * HALF]
        bias = jnp.concatenate([half, half], axis=-1)
        out_ref[gj] = qp + bias


def _build_table(gset_2q, qpair, G2):
    qb = Q // 16
    return pl.pallas_call(
        _table_body,
        grid_spec=pltpu.PrefetchScalarGridSpec(
            num_scalar_prefetch=1,
            grid=(Q // 2 // qb,),
            in_specs=[
                pl.BlockSpec((qb, 2 * DC), lambda q, gset: (q, 0)),
                pl.BlockSpec((16, 1, DC), lambda q, gset: (0, 0, 0)),
            ],
            out_specs=pl.BlockSpec(
                (2 * N2, qb, DC), lambda q, gset: (0, q, 0)
            ),
        ),
        out_shape=jax.ShapeDtypeStruct((2 * N2, Q // 2, DC), jnp.float32),
    )(gset_2q, qpair, G2)


E_PER_W = R2 // NW
CE = 256
NCE = E_PER_W // CE


def _sc_body(table_hbm, lay3_hbm, out_hbm, idx_raw, idx_v, data_v, sem_g, sem_out):
    w = lax.axis_index("s") * NC + lax.axis_index("c")
    g = w // WPG
    ebase = (w % WPG) * E_PER_W
    lane = lax.iota(jnp.int32, 16)
    off = [(2 * g + tc) * Q for tc in range(2)]

    def out_dst(c):
        hb0 = (R1 + g * E + ebase + c * CE) // 64
        return out_hbm.at[pl.ds(hb0, 4)]

    def interleave(i):
        for m in range(4):
            for tc in range(2):
                for s in range(8):
                    vals = idx_raw[4 * i + m, tc, pl.ds(16 * s, 16)] + off[tc]
                    row = 2 * m + (1 if s >= 4 else 0)
                    col = 2 * lane + ((32 * s + tc) % 128)
                    plsc.store_scatter(
                        idx_v, [jnp.full((16,), row, jnp.int32), col], vals
                    )

    m0w = pl.multiple_of(ebase // 128, 64)
    pltpu.sync_copy(lay3_hbm.at[pl.ds(m0w, E_PER_W // 128)], idx_raw)
    interleave(0)

    def pair(i, _):
        c0 = 2 * i
        dsts = [out_dst(c0), out_dst(c0 + 1)]

        @pl.when(c0 >= 2)
        def _():
            for p in range(2):
                pltpu.make_async_copy(
                    data_v.at[p], dsts[p], sem_out.at[p]
                ).wait()

        copies = [
            pltpu.async_copy(
                table_hbm.at[idx_v.at[k]], data_v.at[k // 4, k % 4], sem_g
            )
            for k in range(8)
        ]
        for cp in copies[:4]:
            cp.wait()
        pltpu.async_copy(data_v.at[0], dsts[0], sem_out.at[0])
        for cp in copies[4:]:
            cp.wait()
        pltpu.async_copy(data_v.at[1], dsts[1], sem_out.at[1])
        @pl.when(i + 1 < NCE // 2)
        def _():
            interleave(i + 1)

        return ()

    lax.fori_loop(0, NCE // 2, pair, (), unroll=False)

    for p, c in ((0, NCE - 2), (1, NCE - 1)):
        pltpu.make_async_copy(data_v.at[p], out_dst(c), sem_out.at[p]).wait()


def _sc_gather(table, lay3):
    mesh = plsc.VectorSubcoreMesh(
        core_axis_name="c", subcore_axis_name="s", num_cores=NC, num_subcores=NS
    )
    f = functools.partial(
        pl.kernel,
        out_type=jax.ShapeDtypeStruct((ROWS * DC // (128 * HALF), 128, HALF), jnp.float32),
        mesh=mesh,
        scratch_types=[
            pltpu.VMEM((E_PER_W // 128, 2, 128), jnp.int32),
            pltpu.VMEM((8, 128), jnp.int32),
            pltpu.VMEM((2, 4, 128, HALF), jnp.float32),
            pltpu.SemaphoreType.DMA,
            pltpu.SemaphoreType.DMA((2,)),
        ],
        compiler_params=pltpu.CompilerParams(
            use_tc_tiling_on_sc=False, needs_layout_passes=False
        ),
    )(_sc_body)
    return f(table, lay3)


def _tok1_body(gset1_ref, prev_ref, qub_ref, g1_ref, out_ref):
    del gset1_ref, prev_ref
    out_ref[...] = qub_ref[...] + g1_ref[0]


def _fill_tok1(gset_1q, prev, qubits, G1):
    return pl.pallas_call(
        _tok1_body,
        grid_spec=pltpu.PrefetchScalarGridSpec(
            num_scalar_prefetch=1,
            grid=(2, N1),
            in_specs=[
                pl.BlockSpec(memory_space=pl.ANY),
                pl.BlockSpec((Q // 2, DC), lambda h, g, gset: (h, 0)),
                pl.BlockSpec((1, 1, DC), lambda h, g, gset: (gset[g], 0, 0)),
            ],
            out_specs=pl.BlockSpec(
                (Q // 2, DC), lambda h, g, gset: (2 * g + h, 0)
            ),
        ),
        out_shape=jax.ShapeDtypeStruct((ROWS, DC), jnp.float32),
        input_output_aliases={1: 0},
    )(gset_1q, prev, qubits, G1[:, None, :])


def kernel(gset_1q, gset_2q, qubits, layout, G1, G2):
    qpair = qubits.reshape(Q // 2, 2 * DC)
    table = _build_table(gset_2q, qpair, G2[:, None, :]).reshape(2 * N2 * Q, HALF)
    lay3 = layout.reshape(E // 128, 128, 2).transpose(0, 2, 1)
    out = _sc_gather(table, lay3).reshape(ROWS, DC)
    return _fill_tok1(gset_1q, out, qubits, G1)


# speedup vs baseline: 9.0153x; 1.0313x over previous
import functools

import jax
import jax.numpy as jnp
from jax import lax
from jax.experimental import pallas as pl
from jax.experimental.pallas import tpu as pltpu
from jax.experimental.pallas import tpu_sc as plsc

N1, N2, Q, E, DC = 8, 4, 8192, 65536, 128
HALF = DC // 2
R1 = N1 * Q
R2 = N2 * E
ROWS = R1 + R2
NC, NS = 2, 16
NW = NC * NS
QB = 512

H_TOTAL = 2 * R2
H_PER_W = H_TOTAL // NW
CH = 512
CR = CH // 2
NCHUNK = H_PER_W // CH
WPG = NW // N2


def _table_body(gset2_ref, qpair_ref, g2_ref, out_ref):
    qp = qpair_ref[...]
    for gj in range(2 * N2):
        gi = gset2_ref[gj // 2]
        row = g2_ref[pl.ds(gi, 1), 0]
        j = gj % 2
        half = row[:, j * HALF:(j + 1) * HALF]
        bias = jnp.concatenate([half, half], axis=-1)
        out_ref[gj] = qp + bias


def _build_table(gset_2q, qpair, G2):
    qb = Q // 16
    return pl.pallas_call(
        _table_body,
        grid_spec=pltpu.PrefetchScalarGridSpec(
            num_scalar_prefetch=1,
            grid=(Q // 2 // qb,),
            in_specs=[
                pl.BlockSpec((qb, DC), lambda q, gset: (q, 0)),
                pl.BlockSpec((16, 1, DC), lambda q, gset: (0, 0, 0)),
            ],
            out_specs=pl.BlockSpec(
                (2 * N2, qb, DC), lambda q, gset: (0, q, 0)
            ),
        ),
        out_shape=jax.ShapeDtypeStruct((2 * N2, Q // 2, DC), jnp.float32),
    )(gset_2q, qpair, G2)


E_PER_W = R2 // NW
CE = 256
NCE = E_PER_W // CE


def _sc_body(table_hbm, lay3_hbm, out_hbm, idx_raw, idx_v, data_v, sem_g, sem_out):
    w = lax.axis_index("s") * NC + lax.axis_index("c")
    g = w // WPG
    ebase = (w % WPG) * E_PER_W
    lane = lax.iota(jnp.int32, 16)
    off = [(2 * g + tc) * Q for tc in range(2)]

    def out_dst(c):
        hb0 = (R1 + g * E + ebase + c * CE) // 64
        return out_hbm.at[pl.ds(hb0, 4)]

    def interleave(i):
        for m in range(4):
            for tc in range(2):
                for s in range(8):
                    vals = idx_raw[4 * i + m, tc, pl.ds(16 * s, 16)] + off[tc]
                    row = 2 * m + (1 if s >= 4 else 0)
                    col = 2 * lane + ((32 * s + tc) % 128)
                    plsc.store_scatter(
                        idx_v, [jnp.full((16,), row, jnp.int32), col], vals
                    )

    m0w = pl.multiple_of(ebase // 128, 64)
    pltpu.sync_copy(lay3_hbm.at[pl.ds(m0w, E_PER_W // 128)], idx_raw)
    interleave(0)

    def pair(i, _):
        c0 = 2 * i
        dsts = [out_dst(c0), out_dst(c0 + 1)]

        @pl.when(c0 >= 2)
        def _():
            pltpu.make_async_copy(data_v.at[0], dsts[0], sem_out.at[0]).wait()

        copies = [
            pltpu.async_copy(
                table_hbm.at[idx_v.at[k]], data_v.at[k // 4, k % 4], sem_g
            )
            for k in range(4)
        ]
        @pl.when(c0 >= 2)
        def _():
            pltpu.make_async_copy(data_v.at[1], dsts[1], sem_out.at[1]).wait()

        copies += [
            pltpu.async_copy(
                table_hbm.at[idx_v.at[4 + k]], data_v.at[1, k], sem_g
            )
            for k in range(4)
        ]
        for cp in copies[:4]:
            cp.wait()
        pltpu.async_copy(data_v.at[0], dsts[0], sem_out.at[0])
        for cp in copies[4:]:
            cp.wait()
        pltpu.async_copy(data_v.at[1], dsts[1], sem_out.at[1])
        @pl.when(i + 1 < NCE // 2)
        def _():
            interleave(i + 1)

        return ()

    lax.fori_loop(0, NCE // 2, pair, (), unroll=False)

    for p, c in ((0, NCE - 2), (1, NCE - 1)):
        pltpu.make_async_copy(data_v.at[p], out_dst(c), sem_out.at[p]).wait()


def _sc_gather(table, lay3):
    mesh = plsc.VectorSubcoreMesh(
        core_axis_name="c", subcore_axis_name="s", num_cores=NC, num_subcores=NS
    )
    f = functools.partial(
        pl.kernel,
        out_type=jax.ShapeDtypeStruct((ROWS * DC // (128 * HALF), 128, HALF), jnp.float32),
        mesh=mesh,
        scratch_types=[
            pltpu.VMEM((E_PER_W // 128, 2, 128), jnp.int32),
            pltpu.VMEM((8, 128), jnp.int32),
            pltpu.VMEM((2, 4, 128, HALF), jnp.float32),
            pltpu.SemaphoreType.DMA,
            pltpu.SemaphoreType.DMA((2,)),
        ],
        compiler_params=pltpu.CompilerParams(
            use_tc_tiling_on_sc=False, needs_layout_passes=False
        ),
    )(_sc_body)
    return f(table, lay3)


def _tok1_body(gset1_ref, prev_ref, qub_ref, g1_ref, out_ref):
    del gset1_ref, prev_ref
    out_ref[...] = qub_ref[...] + g1_ref[0]


def _fill_tok1(gset_1q, prev, qubits, G1):
    return pl.pallas_call(
        _tok1_body,
        grid_spec=pltpu.PrefetchScalarGridSpec(
            num_scalar_prefetch=1,
            grid=(N1,),
            in_specs=[
                pl.BlockSpec(memory_space=pl.ANY),
                pl.BlockSpec((Q, DC), lambda g, gset: (0, 0)),
                pl.BlockSpec((1, 1, DC), lambda g, gset: (gset[g], 0, 0)),
            ],
            out_specs=pl.BlockSpec((Q, DC), lambda g, gset: (g, 0)),
        ),
        out_shape=jax.ShapeDtypeStruct((ROWS, DC), jnp.float32),
        input_output_aliases={1: 0},
    )(gset_1q, prev, qubits, G1[:, None, :])


def kernel(gset_1q, gset_2q, qubits, layout, G1, G2):
    qpair = qubits[:, :HALF].reshape(Q // 2, DC)
    table = _build_table(gset_2q, qpair, G2[:, None, :]).reshape(2 * N2 * Q, HALF)
    lay3 = layout.reshape(E // 128, 128, 2).transpose(0, 2, 1)
    out = _sc_gather(table, lay3).reshape(ROWS, DC)
    return _fill_tok1(gset_1q, out, qubits, G1)
